# baseline (speedup 1.0000x reference)
import functools

import jax
import jax.numpy as jnp
from jax import lax
from jax.experimental import pallas as pl
from jax.experimental.pallas import tpu as pltpu
from jax.experimental.pallas import tpu_sc as plsc

_NWORDS = 1000000
_EMB = 64
_CTX = 200
_LANES = 16
_CHUNK = 128
_NCHUNK = 2
_TAIL = _CTX - _CHUNK

_BLK = 8192


_GROUPS = 12
_TAIL8 = 8


def _pick(vec_i32, j):
    return vec_i32[j]


def _sc_body(words_hbm, table_hbm, out_hbm,
             wa, wb, slab_all, row_all, slabs_v, acc_v, *sems):
    cid = lax.axis_index("c")
    sid = lax.axis_index("s")

    @pl.when(jnp.logical_and(cid == 0, sid == 0))
    def _():
        pltpu.sync_copy(words_hbm.at[pl.ds(0, _CHUNK)], wa)
        pltpu.sync_copy(words_hbm.at[pl.ds(_CHUNK, _TAIL)], wb)
        for j in range(_CHUNK // _LANES):
            v = wa[pl.ds(j * _LANES, _LANES)]
            slab_all[pl.ds(j * _LANES, _LANES)] = lax.shift_right_logical(v, 3)
            row_all[pl.ds(j * _LANES, _LANES)] = lax.bitwise_and(v, 7)
        for s in (0, 16, 32, 48, 56):
            v = wb[pl.ds(s, _LANES)]
            slab_all[pl.ds(_CHUNK + s, _LANES)] = lax.shift_right_logical(v, 3)
            row_all[pl.ds(_CHUNK + s, _LANES)] = lax.bitwise_and(v, 7)

        def fire(slab_chunk, j):
            pltpu.async_copy(table_hbm.at[_pick(slab_chunk, j)],
                             slabs_v.at[j], sems[j])

        def drain_acc(row_chunk, j, accs):
            pltpu.make_async_copy(
                table_hbm.at[0], slabs_v.at[j], sems[j]).wait()
            row = _pick(row_chunk, j)
            return [accs[c] + slabs_v[j, row, pl.ds(c * _LANES, _LANES)]
                    for c in range(_EMB // _LANES)]

        sl0 = slab_all[pl.ds(0, _LANES)]
        for j in range(_LANES):
            fire(sl0, j)

        def round_body(g, accs):
            row_chunk = row_all[pl.ds((g - 1) * _LANES, _LANES)]
            slab_chunk = slab_all[pl.ds(g * _LANES, _LANES)]
            accs = list(accs)
            for j in range(_LANES):
                accs = drain_acc(row_chunk, j, accs)
                fire(slab_chunk, j)
            return tuple(accs)

        accs = tuple(jnp.zeros((_LANES,), jnp.float32)
                     for _ in range(_EMB // _LANES))
        accs = lax.fori_loop(1, _GROUPS, round_body, accs)
        accs = list(accs)
        row_chunk = row_all[pl.ds((_GROUPS - 1) * _LANES, _LANES)]
        for j in range(_LANES):
            accs = drain_acc(row_chunk, j, accs)
        tail_slab = slab_all[pl.ds(_CTX - _LANES, _LANES)]
        tail_row = row_all[pl.ds(_CTX - _LANES, _LANES)]
        for j in range(_LANES - _TAIL8, _LANES):
            fire(tail_slab, j)
        for j in range(_LANES - _TAIL8, _LANES):
            accs = drain_acc(tail_row, j, accs)
        for c in range(_EMB // _LANES):
            acc_v[pl.ds(c * _LANES, _LANES)] = accs[c]
        pltpu.sync_copy(acc_v, out_hbm)


@functools.cache
def _emb_sum_sc():
    return pl.kernel(
        _sc_body,
        out_type=jax.ShapeDtypeStruct((_EMB,), jnp.float32),
        mesh=plsc.VectorSubcoreMesh(core_axis_name="c", subcore_axis_name="s"),
        scratch_types=[
            pltpu.VMEM((_CHUNK,), jnp.int32),
            pltpu.VMEM((_TAIL,), jnp.int32),
            pltpu.VMEM((_CTX,), jnp.int32),
            pltpu.VMEM((_CTX,), jnp.int32),
            pltpu.VMEM((_LANES, 8, _EMB), jnp.float32),
            pltpu.VMEM((_EMB,), jnp.float32),
        ] + [pltpu.SemaphoreType.DMA] * _LANES,
    )


def _proj_body(emb_ref, w_ref, b_ref, o_ref):
    o_ref[...] = lax.dot_general(
        emb_ref[...], w_ref[...],
        dimension_numbers=(((1,), (1,)), ((), ())),
        preferred_element_type=jnp.float32,
    ) + b_ref[...][None, :]


def _proj(emb_row, proj_weight, proj_bias):
    grid = pl.cdiv(_NWORDS, _BLK)
    return pl.pallas_call(
        _proj_body,
        grid=(grid,),
        in_specs=[
            pl.BlockSpec((1, _EMB), lambda i: (0, 0)),
            pl.BlockSpec((_BLK, _EMB), lambda i: (i, 0)),
            pl.BlockSpec((_BLK,), lambda i: (i,)),
        ],
        out_specs=pl.BlockSpec((1, _BLK), lambda i: (0, i)),
        out_shape=jax.ShapeDtypeStruct((1, _NWORDS), jnp.float32),
        compiler_params=pltpu.CompilerParams(
            dimension_semantics=("arbitrary",)),
    )(emb_row, proj_weight, proj_bias)


def kernel(words, emb_weight, proj_weight, proj_bias):
    table3 = emb_weight.reshape(_NWORDS // 8, 8, _EMB)
    emb_row = _emb_sum_sc()(words.astype(jnp.int32), table3)
    return _proj(emb_row.reshape(1, _EMB), proj_weight, proj_bias)
